# baseline (speedup 1.0000x reference)
import jax
import jax.numpy as jnp
from jax.experimental import pallas as pl
from jax.experimental.pallas import tpu as pltpu


def _pick_bm(n, target=400):
    best = None
    for bm in range(8, min(n, target) + 1, 8):
        if n % bm == 0:
            best = bm
    return best if best is not None else n


def _prep_kernel(x_ref, w_ref, b_ref, of_ref, ob_ref):
    y1 = (
        jnp.dot(x_ref[...], w_ref[...], preferred_element_type=jnp.float32)
        + b_ref[...]
    )
    of_ref[...] = y1
    ob_ref[...] = y1.astype(jnp.bfloat16)


def _make_fused_kernel(bm, nb, k_stash):
    ilast = nb - 1

    def _fused(y1f_ref, y1b_ref, adj_ref, w2_ref, b2_ref,
               w3_ref, b3_ref, w4_ref, b4_ref,
               emb_ref, z_ref, y2_s, y2f_s, cast_s):
        p = pl.program_id(0)
        i = pl.program_id(1)
        f32 = jnp.float32
        bf16 = jnp.bfloat16
        slot = jnp.minimum(i, k_stash) * bm

        def head(emb):
            emb_ref[...] = emb
            t = jnp.maximum(
                jnp.dot(emb, w3_ref[...], preferred_element_type=f32)
                + b3_ref[...],
                0.0,
            )
            z_ref[...] = (
                jnp.dot(t, w4_ref[...], preferred_element_type=f32)
                + b4_ref[...]
            )

        def layer1_tail(h):
            h = jnp.maximum(h, 0.0)
            y2_s[pl.ds(i * bm, bm), :] = (
                jnp.dot(h, w2_ref[...], preferred_element_type=f32)
                + b2_ref[...]
            ).astype(bf16)

        @pl.when(p == 0)
        def _():
            if k_stash > 0:
                @pl.when(i < k_stash)
                def _():
                    cast_s[pl.ds(slot, bm), :] = adj_ref[...].astype(bf16)
                    tile = cast_s[pl.ds(slot, bm), :]
                    layer1_tail(jnp.dot(tile, y1b_ref[...],
                                        preferred_element_type=f32))

            @pl.when(i >= k_stash)
            def _():
                layer1_tail(jnp.dot(adj_ref[...], y1f_ref[...],
                                    preferred_element_type=f32))

                @pl.when(i == ilast)
                def _():
                    y2f_s[...] = y2_s[...].astype(f32)
                    head(jnp.dot(adj_ref[...], y2f_s[...],
                                 preferred_element_type=f32))

        if k_stash > 0:
            @pl.when(jnp.logical_and(p == 1,
                                     jnp.logical_and(i < k_stash, i < ilast)))
            def _():
                a = cast_s[pl.ds(slot, bm), :]
                head(jnp.dot(a, y2_s[...], preferred_element_type=f32))

        @pl.when(jnp.logical_and(p == 1,
                                 jnp.logical_and(i >= k_stash, i < ilast)))
        def _():
            head(jnp.dot(adj_ref[...], y2f_s[...],
                         preferred_element_type=f32))

    return _fused


@jax.jit
def kernel(x, Adj_, W1, b1, W2, b2, W3, b3, W4, b4):
    n, in_dim = x.shape
    hid = W1.shape[1]
    emb_d = W2.shape[1]
    proj = W4.shape[1]
    f32 = jnp.float32

    b1r = b1.reshape(1, -1)
    b2r = b2.reshape(1, -1)
    b3r = b3.reshape(1, -1)
    b4r = b4.reshape(1, -1)

    y1f, y1b = pl.pallas_call(
        _prep_kernel,
        out_shape=[
            jax.ShapeDtypeStruct((n, hid), f32),
            jax.ShapeDtypeStruct((n, hid), jnp.bfloat16),
        ],
    )(x, W1, b1r)

    bm = _pick_bm(n)
    nb = n // bm
    ilast = nb - 1
    park = max(ilast - 1, 0)
    k_stash = max(0, min(2, nb - 1))
    grid = (2, nb)

    def adj_idx(p, i):
        p1 = jnp.where(i < k_stash, ilast, jnp.where(i == ilast, park, i))
        return (jnp.where(p == 0, i, p1), 0)

    def out_idx(p, i):
        return (jnp.where(p == 0, ilast, jnp.minimum(i, park)), 0)

    const2 = lambda r, c: pl.BlockSpec((r, c), lambda p, i: (0, 0))

    emb, z = pl.pallas_call(
        _make_fused_kernel(bm, nb, k_stash),
        grid=grid,
        in_specs=[
            const2(n, hid),
            const2(n, hid),
            pl.BlockSpec((bm, n), adj_idx),
            const2(hid, emb_d),
            const2(1, emb_d),
            const2(emb_d, proj),
            const2(1, proj),
            const2(proj, proj),
            const2(1, proj),
        ],
        out_specs=[
            pl.BlockSpec((bm, emb_d), out_idx),
            pl.BlockSpec((bm, proj), out_idx),
        ],
        out_shape=[
            jax.ShapeDtypeStruct((n, emb_d), f32),
            jax.ShapeDtypeStruct((n, proj), f32),
        ],
        scratch_shapes=[
            pltpu.VMEM((n, emb_d), jnp.bfloat16),
            pltpu.VMEM((n, emb_d), f32),
            pltpu.VMEM((max(k_stash, 1) * bm, n), jnp.bfloat16),
        ],
        compiler_params=pltpu.CompilerParams(
            dimension_semantics=("arbitrary", "arbitrary"),
            vmem_limit_bytes=64 * 1024 * 1024,
        ),
    )(y1f, y1b, Adj_, W2, b2r, W3, b3r, W4, b4r)

    return (z, emb)
